# baseline (speedup 1.0000x reference)
import functools

import jax
import jax.numpy as jnp
from jax import lax
from jax.experimental import pallas as pl
from jax.experimental.pallas import tpu as pltpu, tpu_sc as plsc

D_IN = 32
D_OUT = 32
NUM_CORES = 2
NUM_SUBCORES = 16
NUM_WORKERS = NUM_CORES * NUM_SUBCORES
LANES = 16

CHUNK = 32
SUP = 128
SUB = SUP // CHUNK
N_BUF = 2


def _make_kernel(n_rows: int):
    assert n_rows % (NUM_WORKERS * SUP * N_BUF) == 0
    b_per_w = n_rows // NUM_WORKERS
    n_sups = b_per_w // SUP
    n_chunks = b_per_w // CHUNK
    mesh = plsc.VectorSubcoreMesh(
        core_axis_name="c", subcore_axis_name="s",
        num_cores=NUM_CORES, num_subcores=NUM_SUBCORES)

    @functools.partial(
        pl.kernel,
        out_type=jax.ShapeDtypeStruct((D_OUT, n_rows), jnp.float32),
        mesh=mesh,
        compiler_params=pltpu.CompilerParams(needs_layout_passes=False),
        scratch_types=[
            pltpu.VMEM((b_per_w,), jnp.int32),
            pltpu.VMEM((N_BUF, CHUNK, D_IN * D_OUT), jnp.float32),
            pltpu.VMEM((N_BUF, D_IN, SUP), jnp.float32),
            pltpu.VMEM((CHUNK * D_IN,), jnp.float32),
            pltpu.VMEM((N_BUF, D_OUT, SUP), jnp.float32),
            pltpu.SemaphoreType.DMA((N_BUF,)),
            pltpu.SemaphoreType.DMA((N_BUF,)),
            pltpu.SemaphoreType.DMA((N_BUF,)),
        ],
    )
    def seg_poly(w_hbm, xt_hbm, idx_hbm, out_hbm,
                 idx_v, w_v, x_t, x_rm, o_t, sem_w, sem_x, sem_o):
        wid = lax.axis_index("s") * NUM_CORES + lax.axis_index("c")
        base = wid * b_per_w

        pltpu.sync_copy(idx_hbm.at[pl.ds(base, b_per_w)], idx_v)

        def issue_x(s, xb):
            pltpu.async_copy(xt_hbm.at[:, pl.ds(base + s * SUP, SUP)],
                             x_t.at[xb], sem_x.at[xb])

        def wait_x(s, xb):
            pltpu.make_async_copy(
                xt_hbm.at[:, pl.ds(base + s * SUP, SUP)],
                x_t.at[xb], sem_x.at[xb]).wait()

        def issue_out(s, ob):
            pltpu.async_copy(o_t.at[ob],
                             out_hbm.at[:, pl.ds(base + s * SUP, SUP)],
                             sem_o.at[ob])

        def wait_out(s, ob):
            pltpu.make_async_copy(
                o_t.at[ob],
                out_hbm.at[:, pl.ds(base + s * SUP, SUP)],
                sem_o.at[ob]).wait()

        def issue_w(k, wb):
            pltpu.async_copy(w_hbm.at[idx_v.at[pl.ds(k * CHUNK, CHUNK)]],
                             w_v.at[wb], sem_w.at[wb])

        def wait_w(k, wb):
            pltpu.make_async_copy(
                w_hbm.at[idx_v.at[pl.ds(k * CHUNK, CHUNK)]],
                w_v.at[wb], sem_w.at[wb]).wait()

        def compute(k, wb, xb, sub):
            ob = xb
            lane = lax.iota(jnp.int32, LANES)
            for i in range(D_IN):
                for rb in range(0, CHUNK, LANES):
                    v = x_t[xb, i, pl.ds(sub * CHUNK + rb, LANES)]
                    plsc.store_scatter(x_rm, [(rb + lane) * D_IN + i], v)

            wait_w(k, wb)

            ob_idx = jnp.full((LANES,), ob, jnp.int32)
            col0 = jnp.full((LANES,), sub * CHUNK, jnp.int32)

            @plsc.parallel_loop(0, CHUNK, unroll=4)
            def row_body(r):
                xv0 = x_rm[pl.ds(r * D_IN, LANES)]
                xv1 = x_rm[pl.ds(r * D_IN + LANES, LANES)]
                acc0 = [None] * 4
                acc1 = [None] * 4
                for i in range(D_IN):
                    xs = xv0[i] if i < LANES else xv1[i - LANES]
                    xb_ = lax.broadcast(xs, (LANES,))
                    t0 = xb_ * w_v[wb, r, pl.ds(i * D_OUT, LANES)]
                    t1 = xb_ * w_v[wb, r, pl.ds(i * D_OUT + LANES, LANES)]
                    j = i % 4
                    acc0[j] = t0 if acc0[j] is None else acc0[j] + t0
                    acc1[j] = t1 if acc1[j] is None else acc1[j] + t1
                col = col0 + r
                plsc.store_scatter(
                    o_t, [ob_idx, lane, col],
                    (acc0[0] + acc0[1]) + (acc0[2] + acc0[3]))
                plsc.store_scatter(
                    o_t, [ob_idx, lane + LANES, col],
                    (acc1[0] + acc1[1]) + (acc1[2] + acc1[3]))

        issue_x(0, 0)
        issue_w(0, 0)

        @pl.loop(0, n_sups, step=N_BUF)
        def sup_loop(s0):
            for sp in range(N_BUF):
                s = s0 + sp
                xb = sp

                @pl.when(s + 1 < n_sups)
                def _():
                    issue_x(s + 1, (sp + 1) % N_BUF)

                wait_x(s, xb)

                @pl.when(s >= N_BUF)
                def _():
                    wait_out(s - N_BUF, xb)

                for sub in range(SUB):
                    k = s * SUB + sub
                    wb = sub % 2

                    @pl.when(k + 1 < n_chunks)
                    def _():
                        issue_w(k + 1, (wb + 1) % 2)

                    compute(k, wb, xb, sub)
                issue_out(s, xb)

        for sp in range(N_BUF):
            wait_out(n_sups - N_BUF + sp, sp)

    return seg_poly


@jax.jit
def kernel(weights, x, weight_indices):
    n_rows = x.shape[0]
    out_t = _make_kernel(n_rows)(weights, x.T, weight_indices)
    return out_t.T


# speedup vs baseline: 1.1304x; 1.1304x over previous
import functools

import jax
import jax.numpy as jnp
from jax import lax
from jax.experimental import pallas as pl
from jax.experimental.pallas import tpu as pltpu, tpu_sc as plsc

D_IN = 32
D_OUT = 32
NUM_CORES = 2
NUM_SUBCORES = 16
NUM_WORKERS = NUM_CORES * NUM_SUBCORES
LANES = 16

CHUNK = 32
SUP = 128
SUB = SUP // CHUNK
N_BUF = 2
XSTR = D_IN + 1
OSTR = SUP + 1


def _make_kernel(n_rows: int):
    assert n_rows % (NUM_WORKERS * SUP * N_BUF) == 0
    b_per_w = n_rows // NUM_WORKERS
    n_sups = b_per_w // SUP
    n_chunks = b_per_w // CHUNK
    mesh = plsc.VectorSubcoreMesh(
        core_axis_name="c", subcore_axis_name="s",
        num_cores=NUM_CORES, num_subcores=NUM_SUBCORES)

    @functools.partial(
        pl.kernel,
        out_type=jax.ShapeDtypeStruct((D_OUT, n_rows), jnp.float32),
        mesh=mesh,
        compiler_params=pltpu.CompilerParams(needs_layout_passes=False),
        scratch_types=[
            pltpu.VMEM((b_per_w,), jnp.int32),
            pltpu.VMEM((N_BUF, CHUNK, D_IN * D_OUT), jnp.float32),
            pltpu.VMEM((N_BUF, D_IN, SUP), jnp.float32),
            pltpu.VMEM((CHUNK * XSTR,), jnp.float32),
            pltpu.VMEM((N_BUF, D_OUT, OSTR), jnp.float32),
            pltpu.SemaphoreType.DMA((N_BUF,)),
            pltpu.SemaphoreType.DMA((N_BUF,)),
            pltpu.SemaphoreType.DMA((N_BUF,)),
        ],
    )
    def seg_poly(w_hbm, xt_hbm, idx_hbm, out_hbm,
                 idx_v, w_v, x_t, x_rm, o_t, sem_w, sem_x, sem_o):
        wid = lax.axis_index("s") * NUM_CORES + lax.axis_index("c")
        base = wid * b_per_w

        pltpu.sync_copy(idx_hbm.at[pl.ds(base, b_per_w)], idx_v)

        def issue_x(s, xb):
            pltpu.async_copy(xt_hbm.at[:, pl.ds(base + s * SUP, SUP)],
                             x_t.at[xb], sem_x.at[xb])

        def wait_x(s, xb):
            pltpu.make_async_copy(
                xt_hbm.at[:, pl.ds(base + s * SUP, SUP)],
                x_t.at[xb], sem_x.at[xb]).wait()

        def issue_out(s, ob):
            pltpu.async_copy(o_t.at[ob, :, pl.ds(0, SUP)],
                             out_hbm.at[:, pl.ds(base + s * SUP, SUP)],
                             sem_o.at[ob])

        def wait_out(s, ob):
            pltpu.make_async_copy(
                o_t.at[ob, :, pl.ds(0, SUP)],
                out_hbm.at[:, pl.ds(base + s * SUP, SUP)],
                sem_o.at[ob]).wait()

        def issue_w(k, wb):
            pltpu.async_copy(w_hbm.at[idx_v.at[pl.ds(k * CHUNK, CHUNK)]],
                             w_v.at[wb], sem_w.at[wb])

        def wait_w(k, wb):
            pltpu.make_async_copy(
                w_hbm.at[idx_v.at[pl.ds(k * CHUNK, CHUNK)]],
                w_v.at[wb], sem_w.at[wb]).wait()

        def compute(k, wb, xb, sub):
            ob = xb
            lane = lax.iota(jnp.int32, LANES)
            for i in range(D_IN):
                for rb in range(0, CHUNK, LANES):
                    v = x_t[xb, i, pl.ds(sub * CHUNK + rb, LANES)]
                    plsc.store_scatter(x_rm, [(rb + lane) * XSTR + i], v)

            wait_w(k, wb)

            ob_idx = jnp.full((LANES,), ob, jnp.int32)
            col0 = jnp.full((LANES,), sub * CHUNK, jnp.int32)

            @plsc.parallel_loop(0, CHUNK, unroll=4)
            def row_body(r):
                xv0 = x_rm[pl.ds(r * XSTR, LANES)]
                xv1 = x_rm[pl.ds(r * XSTR + LANES, LANES)]
                acc0 = [None] * 4
                acc1 = [None] * 4
                for i in range(D_IN):
                    xs = xv0[i] if i < LANES else xv1[i - LANES]
                    xb_ = lax.broadcast(xs, (LANES,))
                    t0 = xb_ * w_v[wb, r, pl.ds(i * D_OUT, LANES)]
                    t1 = xb_ * w_v[wb, r, pl.ds(i * D_OUT + LANES, LANES)]
                    j = i % 4
                    acc0[j] = t0 if acc0[j] is None else acc0[j] + t0
                    acc1[j] = t1 if acc1[j] is None else acc1[j] + t1
                col = col0 + r
                plsc.store_scatter(
                    o_t, [ob_idx, lane, col],
                    (acc0[0] + acc0[1]) + (acc0[2] + acc0[3]))
                plsc.store_scatter(
                    o_t, [ob_idx, lane + LANES, col],
                    (acc1[0] + acc1[1]) + (acc1[2] + acc1[3]))

        issue_x(0, 0)
        issue_w(0, 0)

        @pl.loop(0, n_sups, step=N_BUF)
        def sup_loop(s0):
            for sp in range(N_BUF):
                s = s0 + sp
                xb = sp

                @pl.when(s + 1 < n_sups)
                def _():
                    issue_x(s + 1, (sp + 1) % N_BUF)

                wait_x(s, xb)

                @pl.when(s >= N_BUF)
                def _():
                    wait_out(s - N_BUF, xb)

                for sub in range(SUB):
                    k = s * SUB + sub
                    wb = sub % 2

                    @pl.when(k + 1 < n_chunks)
                    def _():
                        issue_w(k + 1, (wb + 1) % 2)

                    compute(k, wb, xb, sub)
                issue_out(s, xb)

        for sp in range(N_BUF):
            wait_out(n_sups - N_BUF + sp, sp)

    return seg_poly


@jax.jit
def kernel(weights, x, weight_indices):
    n_rows = x.shape[0]
    out_t = _make_kernel(n_rows)(weights, x.T, weight_indices)
    return out_t.T
